# baseline (speedup 1.0000x reference)
import jax
import jax.numpy as jnp
from jax.experimental import pallas as pl
from jax.experimental.pallas import tpu as pltpu

_N_TILE = 512


def _chamfer_kernel(a_ref, bt_ref, fwd_ref, bwd_ref):
    i = pl.program_id(1)
    n_tiles = pl.num_programs(1)
    a = a_ref[0]
    bt = bt_ref[0]

    a2 = jnp.sum(a * a, axis=1, keepdims=True)
    b2 = jnp.sum(bt * bt, axis=0, keepdims=True)

    s = jax.lax.dot_general(
        a, -2.0 * bt, (((1,), (0,)), ((), ())), preferred_element_type=jnp.float32
    )
    e = s + b2
    f = s + a2

    fwd_ref[0, :, :] = jnp.maximum(jnp.min(e, axis=1, keepdims=True) + a2, 0.0)
    bwd_tile = jnp.min(f, axis=0)

    @pl.when(i == 0)
    def _():
        bwd_ref[0, 0, :] = bwd_tile

    @pl.when(i != 0)
    def _():
        bwd_ref[0, 0, :] = jnp.minimum(bwd_ref[0, 0, :], bwd_tile)

    @pl.when(i == n_tiles - 1)
    def _():
        bwd_ref[0, 0, :] = jnp.maximum(bwd_ref[0, 0, :] + b2[0, :], 0.0)


@jax.jit
def kernel(yhat, y):
    B, N, D = yhat.shape
    M = y.shape[1]
    y_t = jnp.transpose(y, (0, 2, 1))
    n_tiles = N // _N_TILE

    fwd, bwd = pl.pallas_call(
        _chamfer_kernel,
        grid=(B, n_tiles),
        in_specs=[
            pl.BlockSpec((1, _N_TILE, D), lambda b, i: (b, i, 0)),
            pl.BlockSpec((1, D, M), lambda b, i: (b, 0, 0)),
        ],
        out_specs=[
            pl.BlockSpec((1, _N_TILE, 1), lambda b, i: (b * n_tiles + i, 0, 0)),
            pl.BlockSpec((1, 1, M), lambda b, i: (b, 0, 0)),
        ],
        out_shape=[
            jax.ShapeDtypeStruct((B * n_tiles, _N_TILE, 1), jnp.float32),
            jax.ShapeDtypeStruct((B, 1, M), jnp.float32),
        ],
        compiler_params=pltpu.CompilerParams(
            dimension_semantics=("parallel", "arbitrary"),
        ),
    )(yhat, y_t)

    loss = jnp.mean(
        jnp.mean(fwd.reshape(B, N), axis=1) + jnp.mean(bwd.reshape(B, M), axis=1)
    )
    return jnp.sqrt(0.5 * loss)


# speedup vs baseline: 1.1466x; 1.1466x over previous
import jax
import jax.numpy as jnp
from jax.experimental import pallas as pl
from jax.experimental.pallas import tpu as pltpu

_N_TILE = 2048


def _chamfer_kernel(a_ref, bt_ref, fwd_ref, bwd_ref):
    i = pl.program_id(1)
    n_tiles = pl.num_programs(1)
    a = a_ref[0]
    bt = bt_ref[0]

    a2 = jnp.sum(a * a, axis=1, keepdims=True)
    b2 = jnp.sum(bt * bt, axis=0, keepdims=True)

    s = jax.lax.dot_general(
        a, -2.0 * bt, (((1,), (0,)), ((), ())), preferred_element_type=jnp.float32
    )
    e = s + b2
    f = s + a2

    fwd_ref[0, :, :] = jnp.maximum(jnp.min(e, axis=1, keepdims=True) + a2, 0.0)
    bwd_tile = jnp.min(f, axis=0)

    @pl.when(i == 0)
    def _():
        bwd_ref[0, 0, :] = bwd_tile

    @pl.when(i != 0)
    def _():
        bwd_ref[0, 0, :] = jnp.minimum(bwd_ref[0, 0, :], bwd_tile)

    @pl.when(i == n_tiles - 1)
    def _():
        bwd_ref[0, 0, :] = jnp.maximum(bwd_ref[0, 0, :] + b2[0, :], 0.0)


@jax.jit
def kernel(yhat, y):
    B, N, D = yhat.shape
    M = y.shape[1]
    y_t = jnp.transpose(y, (0, 2, 1))
    n_tiles = N // _N_TILE

    fwd, bwd = pl.pallas_call(
        _chamfer_kernel,
        grid=(B, n_tiles),
        in_specs=[
            pl.BlockSpec((1, _N_TILE, D), lambda b, i: (b, i, 0)),
            pl.BlockSpec((1, D, M), lambda b, i: (b, 0, 0)),
        ],
        out_specs=[
            pl.BlockSpec((1, _N_TILE, 1), lambda b, i: (b * n_tiles + i, 0, 0)),
            pl.BlockSpec((1, 1, M), lambda b, i: (b, 0, 0)),
        ],
        out_shape=[
            jax.ShapeDtypeStruct((B * n_tiles, _N_TILE, 1), jnp.float32),
            jax.ShapeDtypeStruct((B, 1, M), jnp.float32),
        ],
        compiler_params=pltpu.CompilerParams(
            dimension_semantics=("parallel", "arbitrary"),
        ),
    )(yhat, y_t)

    loss = jnp.mean(
        jnp.mean(fwd.reshape(B, N), axis=1) + jnp.mean(bwd.reshape(B, M), axis=1)
    )
    return jnp.sqrt(0.5 * loss)


# speedup vs baseline: 1.1799x; 1.0290x over previous
import jax
import jax.numpy as jnp
from jax.experimental import pallas as pl
from jax.experimental.pallas import tpu as pltpu

_N_TILE = 4096


def _chamfer_kernel(a_ref, bt_ref, fwd_ref, bwd_ref):
    i = pl.program_id(1)
    n_tiles = pl.num_programs(1)
    a = a_ref[0]
    bt = bt_ref[0]

    a2 = jnp.sum(a * a, axis=1, keepdims=True)
    b2 = jnp.sum(bt * bt, axis=0, keepdims=True)

    s = jax.lax.dot_general(
        a, -2.0 * bt, (((1,), (0,)), ((), ())), preferred_element_type=jnp.float32
    )
    e = s + b2
    f = s + a2

    fwd_ref[0, :, :] = jnp.maximum(jnp.min(e, axis=1, keepdims=True) + a2, 0.0)
    bwd_tile = jnp.min(f, axis=0)

    @pl.when(i == 0)
    def _():
        bwd_ref[0, 0, :] = bwd_tile

    @pl.when(i != 0)
    def _():
        bwd_ref[0, 0, :] = jnp.minimum(bwd_ref[0, 0, :], bwd_tile)

    @pl.when(i == n_tiles - 1)
    def _():
        bwd_ref[0, 0, :] = jnp.maximum(bwd_ref[0, 0, :] + b2[0, :], 0.0)


@jax.jit
def kernel(yhat, y):
    B, N, D = yhat.shape
    M = y.shape[1]
    y_t = jnp.transpose(y, (0, 2, 1))
    n_tiles = N // _N_TILE

    fwd, bwd = pl.pallas_call(
        _chamfer_kernel,
        grid=(B, n_tiles),
        in_specs=[
            pl.BlockSpec((1, _N_TILE, D), lambda b, i: (b, i, 0)),
            pl.BlockSpec((1, D, M), lambda b, i: (b, 0, 0)),
        ],
        out_specs=[
            pl.BlockSpec((1, _N_TILE, 1), lambda b, i: (b * n_tiles + i, 0, 0)),
            pl.BlockSpec((1, 1, M), lambda b, i: (b, 0, 0)),
        ],
        out_shape=[
            jax.ShapeDtypeStruct((B * n_tiles, _N_TILE, 1), jnp.float32),
            jax.ShapeDtypeStruct((B, 1, M), jnp.float32),
        ],
        compiler_params=pltpu.CompilerParams(
            dimension_semantics=("parallel", "arbitrary"),
        ),
    )(yhat, y_t)

    loss = jnp.mean(
        jnp.mean(fwd.reshape(B, N), axis=1) + jnp.mean(bwd.reshape(B, M), axis=1)
    )
    return jnp.sqrt(0.5 * loss)


# speedup vs baseline: 1.1953x; 1.0131x over previous
import jax
import jax.numpy as jnp
from jax.experimental import pallas as pl
from jax.experimental.pallas import tpu as pltpu


def _chamfer_kernel(a_ref, bt_ref, fwd_ref, bwd_ref):
    a = a_ref[0]
    bt = bt_ref[0]

    a2 = jnp.sum(a * a, axis=1, keepdims=True)
    b2 = jnp.sum(bt * bt, axis=0, keepdims=True)

    s = jax.lax.dot_general(
        a, -2.0 * bt, (((1,), (0,)), ((), ())), preferred_element_type=jnp.float32
    )
    e = s + b2
    f = s + a2

    fwd_ref[0, :, :] = jnp.maximum(jnp.min(e, axis=1, keepdims=True) + a2, 0.0)
    bwd_ref[0, 0, :] = jnp.maximum(jnp.min(f, axis=0) + b2[0, :], 0.0)


@jax.jit
def kernel(yhat, y):
    B, N, D = yhat.shape
    M = y.shape[1]
    y_t = jnp.transpose(y, (0, 2, 1))

    fwd, bwd = pl.pallas_call(
        _chamfer_kernel,
        grid=(B,),
        in_specs=[
            pl.BlockSpec((1, N, D), lambda b: (b, 0, 0)),
            pl.BlockSpec((1, D, M), lambda b: (b, 0, 0)),
        ],
        out_specs=[
            pl.BlockSpec((1, N, 1), lambda b: (b, 0, 0)),
            pl.BlockSpec((1, 1, M), lambda b: (b, 0, 0)),
        ],
        out_shape=[
            jax.ShapeDtypeStruct((B, N, 1), jnp.float32),
            jax.ShapeDtypeStruct((B, 1, M), jnp.float32),
        ],
        compiler_params=pltpu.CompilerParams(
            dimension_semantics=("arbitrary",),
        ),
    )(yhat, y_t)

    loss = jnp.mean(
        jnp.mean(fwd.reshape(B, N), axis=1) + jnp.mean(bwd.reshape(B, M), axis=1)
    )
    return jnp.sqrt(0.5 * loss)
